# baseline (speedup 1.0000x reference)
import functools

import jax
import jax.numpy as jnp
from jax import lax
from jax.experimental import pallas as pl
from jax.experimental.pallas import tpu as pltpu
from jax.experimental.pallas import tpu_sc as plsc

N_EMBED = 8192
EMBED_DIM = 3
Z_CHANNELS = 3
L = 16
NW = 32
NCHUNK = 8
NQ = 4


@functools.partial(jax.jit, static_argnames=("n_total",))
def _sc_embed(idx_flat, tableT_flat, w16, b16, n_total):
    b_per_w = n_total // NW
    groups = b_per_w // L
    gpc = groups // NCHUNK
    cpw = b_per_w // NCHUNK
    vq = N_EMBED // NQ
    mesh = plsc.VectorSubcoreMesh(core_axis_name="c", subcore_axis_name="s")

    @functools.partial(
        pl.kernel,
        mesh=mesh,
        out_type=jax.ShapeDtypeStruct((NW, Z_CHANNELS, b_per_w), jnp.float32),
        compiler_params=pltpu.CompilerParams(
            needs_layout_passes=False, use_tc_tiling_on_sc=False),
        scratch_types=[
            pltpu.VMEM((b_per_w,), jnp.int32),
            pltpu.VMEM((N_EMBED * EMBED_DIM,), jnp.float32),
            pltpu.VMEM((N_EMBED * Z_CHANNELS,), jnp.float32),
            pltpu.VMEM((Z_CHANNELS, b_per_w), jnp.float32),
            pltpu.VMEM((L,), jnp.float32),
            pltpu.VMEM((L,), jnp.float32),
            pltpu.VMEM_SHARED((N_EMBED * EMBED_DIM,), jnp.float32),
            pltpu.SemaphoreType.DMA,
            pltpu.SemaphoreType.DMA,
            pltpu.SemaphoreType.DMA,
        ] + [pltpu.SemaphoreType.DMA] * NQ,
    )
    def k(idx_hbm, table_hbm, w16_hbm, b16_hbm, out_hbm,
          idx_v, tab_v, ftab_v, out_v, w16_v, b16_v, tab_s,
          sem_i, sem_w, sem_o, *sem_q):
        sid = lax.axis_index("s")
        wid = sid * 2 + lax.axis_index("c")
        base = wid * b_per_w
        cp_i = pltpu.async_copy(idx_hbm.at[pl.ds(base, b_per_w)], idx_v, sem_i)
        @pl.when(sid == 0)
        def _():
            pltpu.sync_copy(table_hbm, tab_s)
        plsc.subcore_barrier()
        qcps = []
        for q in range(NQ):
            qcps.append([
                pltpu.async_copy(
                    tab_s.at[pl.ds(c * N_EMBED + q * vq, vq)],
                    tab_v.at[pl.ds(c * N_EMBED + q * vq, vq)], sem_q[q])
                for c in range(EMBED_DIM)])
        cp_w1 = pltpu.async_copy(w16_hbm, w16_v, sem_w)
        cp_w2 = pltpu.async_copy(b16_hbm, b16_v, sem_w)
        cp_w1.wait()
        cp_w2.wait()

        def splat(ref, k_):
            return plsc.load_gather(ref, [jnp.full((L,), k_, jnp.int32)])

        W = [[splat(w16_v, 1 + i * 3 + j) for j in range(3)] for i in range(3)]
        b = [splat(b16_v, 1 + j) for j in range(3)]

        def fuse(i):
            p = i * L
            t = [tab_v[pl.ds(c * N_EMBED + p, L)] for c in range(EMBED_DIM)]
            for j in range(Z_CHANNELS):
                ftab_v[pl.ds(j * N_EMBED + p, L)] = (
                    t[0] * W[0][j] + t[1] * W[1][j] + t[2] * W[2][j] + b[j])

        with jax.named_scope("fuse"):
            for q in range(NQ):
                for cp in qcps[q]:
                    cp.wait()
                plsc.parallel_loop(q * vq // L, (q + 1) * vq // L,
                                   unroll=4)(fuse)

        def body(i):
            p = i * L
            rows = idx_v[pl.ds(p, L)]
            for j in range(Z_CHANNELS):
                out_v[j, pl.ds(p, L)] = plsc.load_gather(
                    ftab_v, [rows + j * N_EMBED])

        with jax.named_scope("in_wait"):
            cp_i.wait()
        outcps = []
        with jax.named_scope("compute"):
            for c in range(NCHUNK):
                plsc.parallel_loop(c * gpc, (c + 1) * gpc, unroll=4)(body)
                outcps.append(pltpu.async_copy(
                    out_v.at[:, pl.ds(c * cpw, cpw)],
                    out_hbm.at[wid, :, pl.ds(c * cpw, cpw)], sem_o))
        with jax.named_scope("drain"):
            for cp in outcps:
                cp.wait()

    return k(idx_flat, tableT_flat, w16, b16)


def kernel(x, table, kernel, bias):
    B, H, Wd = x.shape
    n_total = B * H * Wd
    idx_flat = x.reshape(n_total).astype(jnp.int32)
    w16 = jnp.pad(kernel.astype(jnp.float32).reshape(9), (1, 6))
    b16 = jnp.pad(bias.astype(jnp.float32), (1, 12))
    tableT_flat = table.astype(jnp.float32).T.reshape(N_EMBED * EMBED_DIM)
    out = _sc_embed(idx_flat, tableT_flat, w16, b16, n_total)
    return out.reshape(B, Z_CHANNELS, H, Wd).transpose(0, 2, 3, 1)


# speedup vs baseline: 1.0255x; 1.0255x over previous
import functools

import jax
import jax.numpy as jnp
from jax import lax
from jax.experimental import pallas as pl
from jax.experimental.pallas import tpu as pltpu
from jax.experimental.pallas import tpu_sc as plsc

N_EMBED = 8192
EMBED_DIM = 3
Z_CHANNELS = 3
L = 16
NW = 32
NCHUNK = 8
NQ = 4


@functools.partial(jax.jit, static_argnames=("n_total",))
def _sc_embed(idx_flat, tableT_flat, wb, n_total):
    b_per_w = n_total // NW
    groups = b_per_w // L
    gpc = groups // NCHUNK
    cpw = b_per_w // NCHUNK
    vq = N_EMBED // NQ
    mesh = plsc.VectorSubcoreMesh(core_axis_name="c", subcore_axis_name="s")

    @functools.partial(
        pl.kernel,
        mesh=mesh,
        out_type=jax.ShapeDtypeStruct((NW, Z_CHANNELS, b_per_w), jnp.float32),
        compiler_params=pltpu.CompilerParams(
            needs_layout_passes=False, use_tc_tiling_on_sc=False),
        scratch_types=[
            pltpu.VMEM((b_per_w,), jnp.int32),
            pltpu.VMEM((N_EMBED * EMBED_DIM,), jnp.float32),
            pltpu.VMEM((N_EMBED * Z_CHANNELS,), jnp.float32),
            pltpu.VMEM((Z_CHANNELS, b_per_w), jnp.float32),
            pltpu.VMEM((L,), jnp.float32),
            pltpu.VMEM_SHARED((N_EMBED * EMBED_DIM,), jnp.float32),
            pltpu.SemaphoreType.DMA,
            pltpu.SemaphoreType.DMA,
            pltpu.SemaphoreType.DMA,
        ] + [pltpu.SemaphoreType.DMA] * NQ,
    )
    def k(idx_hbm, table_hbm, wb_hbm, out_hbm,
          idx_v, tab_v, ftab_v, out_v, wb_v, tab_s,
          sem_i, sem_w, sem_o, *sem_q):
        sid = lax.axis_index("s")
        wid = sid * 2 + lax.axis_index("c")
        base = wid * b_per_w
        cp_i = pltpu.async_copy(idx_hbm.at[pl.ds(base, b_per_w)], idx_v, sem_i)
        @pl.when(sid == 0)
        def _():
            pltpu.sync_copy(table_hbm, tab_s)
        plsc.subcore_barrier()
        qcps = []
        for q in range(NQ):
            qcps.append([
                pltpu.async_copy(
                    tab_s.at[pl.ds(c * N_EMBED + q * vq, vq)],
                    tab_v.at[pl.ds(c * N_EMBED + q * vq, vq)], sem_q[q])
                for c in range(EMBED_DIM)])
        cp_w = pltpu.async_copy(wb_hbm, wb_v, sem_w)
        cp_w.wait()

        def splat(k_):
            return plsc.load_gather(wb_v, [jnp.full((L,), k_, jnp.int32)])

        W = [[splat(1 + i * 3 + j) for j in range(3)] for i in range(3)]
        b = [splat(10 + j) for j in range(3)]

        def fuse(i):
            p = i * L
            t = [tab_v[pl.ds(c * N_EMBED + p, L)] for c in range(EMBED_DIM)]
            for j in range(Z_CHANNELS):
                ftab_v[pl.ds(j * N_EMBED + p, L)] = (
                    t[0] * W[0][j] + t[1] * W[1][j] + t[2] * W[2][j] + b[j])

        for q in range(NQ):
            for cp in qcps[q]:
                cp.wait()
            plsc.parallel_loop(q * vq // L, (q + 1) * vq // L,
                               unroll=4)(fuse)

        def body(i):
            p = i * L
            rows = idx_v[pl.ds(p, L)]
            for j in range(Z_CHANNELS):
                out_v[j, pl.ds(p, L)] = plsc.load_gather(
                    ftab_v, [rows + j * N_EMBED])

        cp_i.wait()
        outcps = []
        for c in range(NCHUNK):
            plsc.parallel_loop(c * gpc, (c + 1) * gpc, unroll=4)(body)
            outcps.append(pltpu.async_copy(
                out_v.at[:, pl.ds(c * cpw, cpw)],
                out_hbm.at[wid, :, pl.ds(c * cpw, cpw)], sem_o))
        for cp in outcps:
            cp.wait()

    return k(idx_flat, tableT_flat, wb)


def kernel(x, table, kernel, bias):
    B, H, Wd = x.shape
    n_total = B * H * Wd
    idx_flat = x.reshape(n_total).astype(jnp.int32)
    wb = jnp.concatenate(
        [jnp.zeros((1,), jnp.float32),
         kernel.astype(jnp.float32).reshape(9),
         bias.astype(jnp.float32).reshape(3),
         jnp.zeros((3,), jnp.float32)]
    )
    tableT_flat = table.astype(jnp.float32).T.reshape(N_EMBED * EMBED_DIM)
    out = _sc_embed(idx_flat, tableT_flat, wb, n_total)
    return out.reshape(B, Z_CHANNELS, H, Wd).transpose(0, 2, 3, 1)
